# baseline (speedup 1.0000x reference)
import numpy as np
import jax
import jax.numpy as jnp
from jax import lax
from jax.experimental import pallas as pl
from jax.experimental.pallas import tpu as pltpu
from jax.experimental.pallas import tpu_sc as plsc

N = 10000
E = 320000
HID = 128
N_LAYERS = 4
N_CLASSES = 10

NC = 2
NS = 16
NW = NC * NS
CPT = HID // NW

CHE = 2048
E_PAD = 327680
CHUNKS = E_PAD // CHE
N_TC = 10240
BNT = 1280

_MASK_HI = np.int32(-65536)
_MASK_LO = np.int32(0xFFFF)


def _sc_msg_body(hpT_hbm, pk_hbm, filtT_hbm, outT_hbm,
                 idx_v, filt_v, hp_v, agg_v, sem_i, sem_f):
    c = lax.axis_index("c")
    s = lax.axis_index("s")
    wid = c * NS + s

    zero16 = jnp.zeros((16,), jnp.float32)

    pltpu.sync_copy(hpT_hbm.at[wid], hp_v)

    def zcol(w, _):
        agg_v[pl.ds(w * 16, 16)] = zero16
        return 0
    lax.fori_loop(0, CPT * N_TC // 16, zcol, 0, unroll=4)

    def issue_loads(g, b):
        pltpu.async_copy(pk_hbm.at[pl.ds(g * CHE, CHE)], idx_v.at[b],
                         sem_i.at[b])
        pltpu.async_copy(filtT_hbm.at[wid, :, pl.ds(g * CHE, CHE)],
                         filt_v.at[b], sem_f.at[b])

    def wait_loads(g, b):
        pltpu.make_async_copy(pk_hbm.at[pl.ds(g * CHE, CHE)], idx_v.at[b],
                              sem_i.at[b]).wait()
        pltpu.make_async_copy(filtT_hbm.at[wid, :, pl.ds(g * CHE, CHE)],
                              filt_v.at[b], sem_f.at[b]).wait()

    def compute(b):
        @plsc.parallel_loop(0, CHE // 16, unroll=16)
        def _edge_step(i):
            pk = idx_v[b, pl.ds(i * 16, 16)]
            sidx = pk & _MASK_LO
            didx = lax.shift_right_logical(pk, 16)
            for c4 in range(CPT):
                off = np.int32(c4 * N_TC)
                fw = filt_v[b, c4, pl.ds(i * 16, 16)]
                g16 = plsc.load_gather(hp_v, [sidx + off])
                plsc.addupdate_scatter(agg_v, [didx + off], g16 * fw)

    issue_loads(0, 0)
    issue_loads(1, 1)

    def outer(gg, _):
        for b in range(2):
            g = gg * 2 + b
            wait_loads(g, b)
            compute(b)

            @pl.when(g + 2 < CHUNKS)
            def _():
                issue_loads(g + 2, b)
        return 0

    lax.fori_loop(0, CHUNKS // 2, outer, 0)

    pltpu.sync_copy(agg_v, outT_hbm.at[wid])


@jax.jit
def _sc_msg(hpT3, pk, filtT3):
    kern = pl.kernel(
        _sc_msg_body,
        out_type=jax.ShapeDtypeStruct((NW, CPT * N_TC), jnp.float32),
        mesh=plsc.VectorSubcoreMesh(core_axis_name="c", subcore_axis_name="s"),
        compiler_params=pltpu.CompilerParams(needs_layout_passes=False),
        scratch_types=[
            pltpu.VMEM((2, CHE), jnp.int32),
            pltpu.VMEM((2, CPT, CHE), jnp.float32),
            pltpu.VMEM((CPT * N_TC,), jnp.float32),
            pltpu.VMEM((CPT * N_TC,), jnp.float32),
            pltpu.SemaphoreType.DMA((2,)),
            pltpu.SemaphoreType.DMA((2,)),
        ],
    )
    return kern(hpT3, pk, filtT3)


BE = 2048


def _filt_body(dist_ref, wf1_ref, bf1_ref, wf2_ref, bf2_ref, out_ref):
    d = dist_ref[0, 0, :]
    centers = lax.broadcasted_iota(jnp.int32, (BE, HID), 1).astype(
        jnp.float32) * (1.0 / (HID - 1))
    dd = d[:, None] - centers
    bf = jnp.exp(dd * dd * (-float(HID)))
    cut = 0.5 * (jnp.cos(jnp.pi * jnp.clip(d, 0.0, 1.0)) + 1.0)
    bf = bf * cut[:, None]
    t = bf @ wf1_ref[...] + bf1_ref[...]
    t = jnp.logaddexp(t, 0.0)
    f = t @ wf2_ref[...] + bf2_ref[...]
    out_ref[...] = f.T


def _filt(dist3d, Wf1, bf1, Wf2, bf2):
    nb = E_PAD // BE
    return pl.pallas_call(
        _filt_body,
        grid=(nb,),
        in_specs=[
            pl.BlockSpec((1, 1, BE), lambda i: (i, 0, 0)),
            pl.BlockSpec((HID, HID), lambda i: (0, 0)),
            pl.BlockSpec((1, HID), lambda i: (0, 0)),
            pl.BlockSpec((HID, HID), lambda i: (0, 0)),
            pl.BlockSpec((1, HID), lambda i: (0, 0)),
        ],
        out_specs=pl.BlockSpec((HID, BE), lambda i: (0, i)),
        out_shape=jax.ShapeDtypeStruct((HID, E_PAD), jnp.float32),
    )(dist3d, Wf1, bf1, Wf2, bf2)


def _mmT_body(w_ref, xT_ref, out_ref):
    out_ref[...] = jnp.dot(w_ref[...].T, xT_ref[...])


def _mmT(w, xT):
    return pl.pallas_call(
        _mmT_body,
        grid=(N_TC // BNT,),
        in_specs=[
            pl.BlockSpec((HID, HID), lambda i: (0, 0)),
            pl.BlockSpec((HID, BNT), lambda i: (0, i)),
        ],
        out_specs=pl.BlockSpec((HID, BNT), lambda i: (0, i)),
        out_shape=jax.ShapeDtypeStruct((HID, N_TC), jnp.float32),
    )(w, xT)


def _denseT_body(a_ref, w2_ref, b2_ref, w1n_ref, out_ref):
    h = jnp.maximum(jnp.dot(w2_ref[...].T, a_ref[...]) + b2_ref[...],
                    0.0)
    out_ref[...] = jnp.dot(w1n_ref[...].T, h)


def _denseT(aT, W2i, b2i, W1n):
    return pl.pallas_call(
        _denseT_body,
        grid=(N_TC // BNT,),
        in_specs=[
            pl.BlockSpec((HID, BNT), lambda i: (0, i)),
            pl.BlockSpec((HID, HID), lambda i: (0, 0)),
            pl.BlockSpec((HID, 1), lambda i: (0, 0)),
            pl.BlockSpec((HID, HID), lambda i: (0, 0)),
        ],
        out_specs=pl.BlockSpec((HID, BNT), lambda i: (0, i)),
        out_shape=jax.ShapeDtypeStruct((HID, N_TC), jnp.float32),
    )(aT, W2i, b2i, W1n)


def _finalT_body(a_ref, w2_ref, b2_ref, wfc_ref, bfc_ref, out_ref, acc_ref):
    i = pl.program_id(0)
    h = jnp.maximum(jnp.dot(w2_ref[...].T, a_ref[...]) + b2_ref[...],
                    0.0)
    col = lax.broadcasted_iota(jnp.int32, (HID, BNT), 1) + i * BNT
    h = jnp.where(col < N, h, 0.0)
    rowsum = jnp.sum(h, axis=1, keepdims=True)

    @pl.when(i == 0)
    def _():
        acc_ref[...] = rowsum

    @pl.when(i > 0)
    def _():
        acc_ref[...] = acc_ref[...] + rowsum

    @pl.when(i == pl.num_programs(0) - 1)
    def _():
        g = acc_ref[...].T * (1.0 / N)
        logits = g @ wfc_ref[...] + bfc_ref[...]
        m = jnp.max(logits, axis=1, keepdims=True)
        z = logits - m
        lse = jnp.log(jnp.sum(jnp.exp(z), axis=1, keepdims=True))
        out_ref[...] = z - lse


def _finalT(aT, W2i, b2i, Wfc, bfc):
    return pl.pallas_call(
        _finalT_body,
        grid=(N_TC // BNT,),
        in_specs=[
            pl.BlockSpec((HID, BNT), lambda i: (0, i)),
            pl.BlockSpec((HID, HID), lambda i: (0, 0)),
            pl.BlockSpec((HID, 1), lambda i: (0, 0)),
            pl.BlockSpec((HID, N_CLASSES), lambda i: (0, 0)),
            pl.BlockSpec((1, N_CLASSES), lambda i: (0, 0)),
        ],
        out_specs=pl.BlockSpec((1, N_CLASSES), lambda i: (0, 0)),
        out_shape=jax.ShapeDtypeStruct((1, N_CLASSES), jnp.float32),
        scratch_shapes=[pltpu.VMEM((HID, 1), jnp.float32)],
    )(aT, W2i, b2i, Wfc, bfc)


def kernel(x, edge_index, edge_dist, W1, W2, b2, Wf1, bf1, Wf2, bf2, Wfc, bfc):
    src = edge_index[0]
    dst = edge_index[1]
    pad = E_PAD - E
    src_p = jnp.concatenate([src, jnp.zeros((pad,), jnp.int32)])
    dst_p = jnp.concatenate([dst, jnp.full((pad,), N, jnp.int32)])
    pk = src_p + dst_p * 65536
    dist_p = jnp.concatenate([edge_dist, jnp.zeros((pad,), jnp.float32)])
    dist3d = dist_p.reshape(E_PAD // BE, 1, BE)

    filtT3 = _filt(dist3d, Wf1, bf1.reshape(1, HID), Wf2,
                   bf2.reshape(1, HID)).reshape(NW, CPT, E_PAD)

    xT = jnp.pad(x.T, ((0, 0), (0, N_TC - N)))
    hpT = _mmT(W1[0], xT)
    b2c = b2.reshape(N_LAYERS, HID, 1)
    for i in range(N_LAYERS):
        aggT3 = _sc_msg(hpT.reshape(NW, CPT * N_TC), pk, filtT3)
        aT = aggT3.reshape(HID, N_TC)
        if i < N_LAYERS - 1:
            hpT = _denseT(aT, W2[i], b2c[i], W1[i + 1])
        else:
            out = _finalT(aT, W2[i], b2c[i], Wfc, bfc.reshape(1, N_CLASSES))
    return out


# speedup vs baseline: 1.1261x; 1.1261x over previous
import numpy as np
import jax
import jax.numpy as jnp
from jax import lax
from jax.experimental import pallas as pl
from jax.experimental.pallas import tpu as pltpu
from jax.experimental.pallas import tpu_sc as plsc

N = 10000
E = 320000
HID = 128
N_LAYERS = 4
N_CLASSES = 10

NC = 2
NS = 16
NW = NC * NS
CPT = HID // NW

CHE = 2048
E_PAD = 327680
CHUNKS = E_PAD // CHE
N_TC = 10240
BNT = 1280

_MASK_HI = np.int32(-65536)
_MASK_LO = np.int32(0xFFFF)


def _sc_msg_body(hpT_hbm, pk_hbm, filtT_hbm, outT_hbm,
                 idx_v, filt_v, hp_v, agg_v, sem_i, sem_f):
    c = lax.axis_index("c")
    s = lax.axis_index("s")
    wid = c * NS + s

    zero16 = jnp.zeros((16,), jnp.float32)

    pltpu.sync_copy(hpT_hbm.at[wid], hp_v)

    def zcol(w, _):
        agg_v[pl.ds(w * 16, 16)] = zero16
        return 0
    lax.fori_loop(0, CPT * N_TC // 16, zcol, 0, unroll=4)

    def issue_loads(g, b):
        pltpu.async_copy(pk_hbm.at[pl.ds(g * CHE, CHE)], idx_v.at[b],
                         sem_i.at[b])
        pltpu.async_copy(filtT_hbm.at[wid, :, pl.ds(g * (CHE // 2),
                                                    CHE // 2)],
                         filt_v.at[b], sem_f.at[b])

    def wait_loads(g, b):
        pltpu.make_async_copy(pk_hbm.at[pl.ds(g * CHE, CHE)], idx_v.at[b],
                              sem_i.at[b]).wait()
        pltpu.make_async_copy(filtT_hbm.at[wid, :, pl.ds(g * (CHE // 2),
                                                         CHE // 2)],
                              filt_v.at[b], sem_f.at[b]).wait()

    def compute(b):
        @plsc.parallel_loop(0, CHE // 32, unroll=4)
        def _edge_step(i):
            pkA = idx_v[b, pl.ds(i * 32, 16)]
            pkB = idx_v[b, pl.ds(i * 32 + 16, 16)]
            sA = pkA & _MASK_LO
            dA = lax.shift_right_logical(pkA, 16)
            sB = pkB & _MASK_LO
            dB = lax.shift_right_logical(pkB, 16)
            for c4 in range(CPT):
                off = np.int32(c4 * N_TC)
                fw = filt_v[b, c4, pl.ds(i * 16, 16)]
                fa = jax.lax.bitcast_convert_type(fw << 16, jnp.float32)
                fb = jax.lax.bitcast_convert_type(fw & _MASK_HI,
                                                  jnp.float32)
                gA = plsc.load_gather(hp_v, [sA + off])
                gB = plsc.load_gather(hp_v, [sB + off])
                plsc.addupdate_scatter(agg_v, [dA + off], gA * fa)
                plsc.addupdate_scatter(agg_v, [dB + off], gB * fb)

    issue_loads(0, 0)
    issue_loads(1, 1)

    def outer(gg, _):
        for b in range(2):
            g = gg * 2 + b
            wait_loads(g, b)
            compute(b)

            @pl.when(g + 2 < CHUNKS)
            def _():
                issue_loads(g + 2, b)
        return 0

    lax.fori_loop(0, CHUNKS // 2, outer, 0)

    pltpu.sync_copy(agg_v, outT_hbm.at[wid])


@jax.jit
def _sc_msg(hpT3, pk, filtT3):
    kern = pl.kernel(
        _sc_msg_body,
        out_type=jax.ShapeDtypeStruct((NW, CPT * N_TC), jnp.float32),
        mesh=plsc.VectorSubcoreMesh(core_axis_name="c", subcore_axis_name="s"),
        compiler_params=pltpu.CompilerParams(needs_layout_passes=False),
        scratch_types=[
            pltpu.VMEM((2, CHE), jnp.int32),
            pltpu.VMEM((2, CPT, CHE // 2), jnp.int32),
            pltpu.VMEM((CPT * N_TC,), jnp.float32),
            pltpu.VMEM((CPT * N_TC,), jnp.float32),
            pltpu.SemaphoreType.DMA((2,)),
            pltpu.SemaphoreType.DMA((2,)),
        ],
    )
    return kern(hpT3, pk, filtT3)


BE = 2048


def _filt_body(dist_ref, wf1_ref, bf1_ref, wf2_ref, bf2_ref, out_ref):
    d = dist_ref[0, 0, :]
    centers = lax.broadcasted_iota(jnp.int32, (BE, HID), 1).astype(
        jnp.float32) * (1.0 / (HID - 1))
    dd = d[:, None] - centers
    bf = jnp.exp(dd * dd * (-float(HID)))
    cut = 0.5 * (jnp.cos(jnp.pi * jnp.clip(d, 0.0, 1.0)) + 1.0)
    bf = bf * cut[:, None]
    t = bf @ wf1_ref[...] + bf1_ref[...]
    t = jnp.logaddexp(t, 0.0)
    f = t @ wf2_ref[...] + bf2_ref[...]
    f3 = f.reshape(BE // 2, 2, HID)
    fe = f3[:, 0, :]
    fo = f3[:, 1, :]
    we = jax.lax.bitcast_convert_type(fe.astype(jnp.bfloat16),
                                      jnp.int16).astype(jnp.int32)
    wo = jax.lax.bitcast_convert_type(fo.astype(jnp.bfloat16),
                                      jnp.int16).astype(jnp.int32)
    out_ref[...] = ((we & 0xFFFF) | (wo << 16)).T


def _filt(dist3d, Wf1, bf1, Wf2, bf2):
    nb = E_PAD // BE
    return pl.pallas_call(
        _filt_body,
        grid=(nb,),
        in_specs=[
            pl.BlockSpec((1, 1, BE), lambda i: (i, 0, 0)),
            pl.BlockSpec((HID, HID), lambda i: (0, 0)),
            pl.BlockSpec((1, HID), lambda i: (0, 0)),
            pl.BlockSpec((HID, HID), lambda i: (0, 0)),
            pl.BlockSpec((1, HID), lambda i: (0, 0)),
        ],
        out_specs=pl.BlockSpec((HID, BE // 2), lambda i: (0, i)),
        out_shape=jax.ShapeDtypeStruct((HID, E_PAD // 2), jnp.int32),
    )(dist3d, Wf1, bf1, Wf2, bf2)


def _mmT_body(w_ref, xT_ref, out_ref):
    out_ref[...] = jnp.dot(w_ref[...].T, xT_ref[...])


def _mmT(w, xT):
    return pl.pallas_call(
        _mmT_body,
        grid=(N_TC // BNT,),
        in_specs=[
            pl.BlockSpec((HID, HID), lambda i: (0, 0)),
            pl.BlockSpec((HID, BNT), lambda i: (0, i)),
        ],
        out_specs=pl.BlockSpec((HID, BNT), lambda i: (0, i)),
        out_shape=jax.ShapeDtypeStruct((HID, N_TC), jnp.float32),
    )(w, xT)


def _denseT_body(a_ref, w2_ref, b2_ref, w1n_ref, out_ref):
    h = jnp.maximum(jnp.dot(w2_ref[...].T, a_ref[...]) + b2_ref[...],
                    0.0)
    out_ref[...] = jnp.dot(w1n_ref[...].T, h)


def _denseT(aT, W2i, b2i, W1n):
    return pl.pallas_call(
        _denseT_body,
        grid=(N_TC // BNT,),
        in_specs=[
            pl.BlockSpec((HID, BNT), lambda i: (0, i)),
            pl.BlockSpec((HID, HID), lambda i: (0, 0)),
            pl.BlockSpec((HID, 1), lambda i: (0, 0)),
            pl.BlockSpec((HID, HID), lambda i: (0, 0)),
        ],
        out_specs=pl.BlockSpec((HID, BNT), lambda i: (0, i)),
        out_shape=jax.ShapeDtypeStruct((HID, N_TC), jnp.float32),
    )(aT, W2i, b2i, W1n)


def _finalT_body(a_ref, w2_ref, b2_ref, wfc_ref, bfc_ref, out_ref, acc_ref):
    i = pl.program_id(0)
    h = jnp.maximum(jnp.dot(w2_ref[...].T, a_ref[...]) + b2_ref[...],
                    0.0)
    col = lax.broadcasted_iota(jnp.int32, (HID, BNT), 1) + i * BNT
    h = jnp.where(col < N, h, 0.0)
    rowsum = jnp.sum(h, axis=1, keepdims=True)

    @pl.when(i == 0)
    def _():
        acc_ref[...] = rowsum

    @pl.when(i > 0)
    def _():
        acc_ref[...] = acc_ref[...] + rowsum

    @pl.when(i == pl.num_programs(0) - 1)
    def _():
        g = acc_ref[...].T * (1.0 / N)
        logits = g @ wfc_ref[...] + bfc_ref[...]
        m = jnp.max(logits, axis=1, keepdims=True)
        z = logits - m
        lse = jnp.log(jnp.sum(jnp.exp(z), axis=1, keepdims=True))
        out_ref[...] = z - lse


def _finalT(aT, W2i, b2i, Wfc, bfc):
    return pl.pallas_call(
        _finalT_body,
        grid=(N_TC // BNT,),
        in_specs=[
            pl.BlockSpec((HID, BNT), lambda i: (0, i)),
            pl.BlockSpec((HID, HID), lambda i: (0, 0)),
            pl.BlockSpec((HID, 1), lambda i: (0, 0)),
            pl.BlockSpec((HID, N_CLASSES), lambda i: (0, 0)),
            pl.BlockSpec((1, N_CLASSES), lambda i: (0, 0)),
        ],
        out_specs=pl.BlockSpec((1, N_CLASSES), lambda i: (0, 0)),
        out_shape=jax.ShapeDtypeStruct((1, N_CLASSES), jnp.float32),
        scratch_shapes=[pltpu.VMEM((HID, 1), jnp.float32)],
    )(aT, W2i, b2i, Wfc, bfc)


def kernel(x, edge_index, edge_dist, W1, W2, b2, Wf1, bf1, Wf2, bf2, Wfc, bfc):
    src = edge_index[0]
    dst = edge_index[1]
    pad = E_PAD - E
    src_p = jnp.concatenate([src, jnp.zeros((pad,), jnp.int32)])
    dst_p = jnp.concatenate([dst, jnp.full((pad,), N, jnp.int32)])
    pk = src_p + dst_p * 65536
    pk = pk.reshape(E_PAD // 32, 16, 2).transpose(0, 2, 1).reshape(E_PAD)
    dist_p = jnp.concatenate([edge_dist, jnp.zeros((pad,), jnp.float32)])
    dist3d = dist_p.reshape(E_PAD // BE, 1, BE)

    filtT3 = _filt(dist3d, Wf1, bf1.reshape(1, HID), Wf2,
                   bf2.reshape(1, HID)).reshape(NW, CPT, E_PAD // 2)

    xT = jnp.pad(x.T, ((0, 0), (0, N_TC - N)))
    hpT = _mmT(W1[0], xT)
    b2c = b2.reshape(N_LAYERS, HID, 1)
    for i in range(N_LAYERS):
        aggT3 = _sc_msg(hpT.reshape(NW, CPT * N_TC), pk, filtT3)
        aT = aggT3.reshape(HID, N_TC)
        if i < N_LAYERS - 1:
            hpT = _denseT(aT, W2[i], b2c[i], W1[i + 1])
        else:
            out = _finalT(aT, W2[i], b2c[i], Wfc, bfc.reshape(1, N_CLASSES))
    return out
